# baseline (speedup 1.0000x reference)
import functools

import jax
import jax.numpy as jnp
from jax import lax
from jax.experimental import pallas as pl
from jax.experimental.pallas import tpu as pltpu
from jax.experimental.pallas import tpu_sc as plsc

NUM_NEIGH = 16
RBF_BINS = 16
PAIR_SIZE = 64
LOCAL_SIZE = 128


def _topk_body(pos_ref, posT_ref, cols_ref, rows_ref, nbr_ref, wrap_ref,
               dsel_ref, *, n_total):
    xi = pos_ref[:, 0:1]
    yi = pos_ref[:, 1:2]
    zi = pos_ref[:, 2:3]
    xj = posT_ref[0:1, :]
    yj = posT_ref[1:2, :]
    zj = posT_ref[2:3, :]
    d2 = jnp.maximum((xi - xj) ** 2 + (yi - yj) ** 2 + (zi - zj) ** 2, 1e-12)

    isaa_j = cols_ref[0:1, :] > 0
    mask_j = cols_ref[1:2, :] > 0
    mask_i = rows_ref[:, 4:5] > 0

    inf = jnp.float32(jnp.inf)
    score_a = jnp.where(mask_i & mask_j & isaa_j, d2, inf)
    score_b = jnp.where(mask_i & mask_j & (~isaa_j), d2, inf)
    r = d2.shape[0]
    iota = lax.broadcasted_iota(jnp.int32, (r, n_total), 1)
    d2_last = d2[:, n_total - 1:n_total]

    for s, score in ((0, score_a), (1, score_b)):
        nbrs, wraps, dsels = [], [], []
        for _ in range(NUM_NEIGH):
            m = jnp.min(score, axis=1, keepdims=True)
            valid = m < inf
            idx = jnp.min(jnp.where(score == m, iota, n_total), axis=1,
                          keepdims=True)
            nbrs.append(jnp.where(valid, idx, -1))
            wraps.append(jnp.where(valid, idx, n_total - 1))
            dsels.append(jnp.where(valid, m, d2_last))
            score = jnp.where(iota == idx, inf, score)
        c0 = s * NUM_NEIGH
        nbr_ref[:, c0:c0 + NUM_NEIGH] = jnp.concatenate(nbrs, axis=1)
        wrap_ref[:, c0:c0 + NUM_NEIGH] = jnp.concatenate(wraps, axis=1)
        dsel_ref[:, c0:c0 + NUM_NEIGH] = jnp.sqrt(
            jnp.concatenate(dsels, axis=1))


def _run_topk(pos, posT, cols_aux, rows_aux, n, r):
    k2 = 2 * NUM_NEIGH
    grid = n // r
    return pl.pallas_call(
        functools.partial(_topk_body, n_total=n),
        grid=(grid,),
        in_specs=[
            pl.BlockSpec((r, 3), lambda i: (i, 0)),
            pl.BlockSpec((3, n), lambda i: (0, 0)),
            pl.BlockSpec((2, n), lambda i: (0, 0)),
            pl.BlockSpec((r, 8), lambda i: (i, 0)),
        ],
        out_specs=[
            pl.BlockSpec((r, k2), lambda i: (i, 0)),
            pl.BlockSpec((r, k2), lambda i: (i, 0)),
            pl.BlockSpec((r, k2), lambda i: (i, 0)),
        ],
        out_shape=[
            jax.ShapeDtypeStruct((n, k2), jnp.int32),
            jax.ShapeDtypeStruct((n, k2), jnp.int32),
            jax.ShapeDtypeStruct((n, k2), jnp.float32),
        ],
    )(pos, posT, cols_aux, rows_aux)


def _gather_packed(table, idx_flat):
    info = plsc.get_sparse_core_info()
    nw = info.num_cores * info.num_subcores
    b = idx_flat.shape[0]
    bw = b // nw
    n = table.shape[0]
    mesh = plsc.VectorSubcoreMesh(core_axis_name="c", subcore_axis_name="s")

    @functools.partial(
        pl.kernel,
        mesh=mesh,
        compiler_params=pltpu.CompilerParams(needs_layout_passes=False),
        out_type=jax.ShapeDtypeStruct((b,), jnp.int32),
        scratch_types=[
            pltpu.VMEM((n,), jnp.int32),
            pltpu.VMEM((bw,), jnp.int32),
            pltpu.VMEM((bw,), jnp.int32),
        ],
    )
    def gk(table_hbm, idx_hbm, out_hbm, table_v, idx_v, out_v):
        wid = lax.axis_index("s") * info.num_cores + lax.axis_index("c")
        base = wid * bw
        pltpu.sync_copy(table_hbm, table_v)
        pltpu.sync_copy(idx_hbm.at[pl.ds(base, bw)], idx_v)

        def body(i, carry):
            off = i * 16
            vec = idx_v[pl.ds(off, 16)]
            out_v[pl.ds(off, 16)] = plsc.load_gather(table_v, [vec])
            return carry

        lax.fori_loop(0, bw // 16, body, 0)
        pltpu.sync_copy(out_v, out_hbm.at[pl.ds(base, bw)])

    return gk(table, idx_flat)


def _pair_body(dsel_ref, packed_ref, nbr_ref, rows_ref, wp_ref, lps_ref,
               lpo_ref, w1_ref, b1_ref, w2_ref, b2_ref, wl_ref, lls_ref,
               llo_ref, pair_ref, local_ref, *, r):
    k2 = 2 * NUM_NEIGH

    chain_i = rows_ref[:, 0:1]
    resid_i = rows_ref[:, 1:2]
    aa_i = rows_ref[:, 2:3]
    isaa_i = rows_ref[:, 3:4]

    step = jnp.float32(20.0 / 15.0)
    centers = jnp.float32(2.0) + lax.broadcasted_iota(
        jnp.int32, (1, RBF_BINS), 1).astype(jnp.float32) * step
    sigma = jnp.float32(20.0 / 16.0)

    pj = packed_ref[...]
    isaa_j = (pj & 1).astype(jnp.float32)
    chain_j = (pj >> 1) & 7
    resid_j = pj >> 4
    same_chain = chain_i == chain_j
    other_chain = (~same_chain).astype(jnp.float32)
    same_res = (same_chain & (resid_i == resid_j)).astype(jnp.float32)
    nbm = (nbr_ref[...] != -1).astype(jnp.float32)

    d_rep = jnp.concatenate(
        [jnp.broadcast_to(dsel_ref[:, k:k + 1], (r, RBF_BINS))
         for k in range(k2)], axis=1)
    centers_rep = jnp.concatenate([centers] * k2, axis=1)
    rbf = jnp.exp(-((d_rep - centers_rep) ** 2) / (2.0 * sigma * sigma))

    rbf_stack = jnp.concatenate(
        [rbf[:, k * RBF_BINS:(k + 1) * RBF_BINS] for k in range(k2)], axis=0)
    col = lambda a: jnp.concatenate(
        [a[:, k:k + 1] for k in range(k2)], axis=0)
    isaa_s, same_s, other_s, nb = (col(isaa_j), col(same_res),
                                   col(other_chain), col(nbm))

    pair = (
        jnp.dot(rbf_stack, wp_ref[0:RBF_BINS, :],
                preferred_element_type=jnp.float32)
        + isaa_s * wp_ref[RBF_BINS:RBF_BINS + 1, :]
        + same_s * wp_ref[RBF_BINS + 1:RBF_BINS + 2, :]
        + other_s * wp_ref[RBF_BINS + 2:RBF_BINS + 3, :])
    mean = jnp.mean(pair, axis=1, keepdims=True)
    var = jnp.mean((pair - mean) ** 2, axis=1, keepdims=True)
    pair = (pair - mean) / jnp.sqrt(var + 1e-5) * lps_ref[...] + lpo_ref[...]
    pair_ref[...] = pair.reshape(k2, r, PAIR_SIZE)

    h = jnp.maximum(
        jnp.dot(pair, w1_ref[...], preferred_element_type=jnp.float32)
        + b1_ref[...], 0.0)
    h = jnp.dot(h, w2_ref[...], preferred_element_type=jnp.float32) + b2_ref[...]
    h = h * nb
    pw = jnp.sum(h.reshape(k2, r, LOCAL_SIZE), axis=0)

    onehot = (lax.broadcasted_iota(jnp.int32, (r, 21), 1) == aa_i).astype(
        jnp.float32)
    lfeat = jnp.concatenate([pw, isaa_i.astype(jnp.float32), onehot], axis=1)
    loc = jnp.dot(lfeat, wl_ref[...], preferred_element_type=jnp.float32)
    mean = jnp.mean(loc, axis=1, keepdims=True)
    var = jnp.mean((loc - mean) ** 2, axis=1, keepdims=True)
    loc = (loc - mean) / jnp.sqrt(var + 1e-5) * lls_ref[...] + llo_ref[...]
    local_ref[...] = loc


def _run_pair(dsel, packed, nbr, rows_aux, W_pair, lps, lpo, W1, b1, W2, b2,
              Wl, lls, llo, n, r):
    k2 = 2 * NUM_NEIGH
    grid = n // r
    full = lambda shape: pl.BlockSpec(shape, lambda i: tuple(0 for _ in shape))
    blk = lambda shape: pl.BlockSpec(shape, lambda i: (i, 0))
    return pl.pallas_call(
        functools.partial(_pair_body, r=r),
        grid=(grid,),
        in_specs=[
            blk((r, k2)), blk((r, k2)), blk((r, k2)), blk((r, 8)),
            full(W_pair.shape), full(lps.shape), full(lpo.shape),
            full(W1.shape), full(b1.shape), full(W2.shape), full(b2.shape),
            full(Wl.shape), full(lls.shape), full(llo.shape),
        ],
        out_specs=[
            pl.BlockSpec((k2, r, PAIR_SIZE), lambda i: (0, i, 0)),
            pl.BlockSpec((r, PAIR_SIZE), lambda i: (i, 0)),
        ],
        out_shape=[
            jax.ShapeDtypeStruct((k2, n, PAIR_SIZE), jnp.float32),
            jax.ShapeDtypeStruct((n, PAIR_SIZE), jnp.float32),
        ],
    )(dsel, packed, nbr, rows_aux, W_pair, lps, lpo, W1, b1, W2, b2,
      Wl, lls, llo)


def kernel(aa, chain_index, residue_index, is_aa, all_atom_positions,
           all_atom_mask, W_pair, ln_pair_scale, ln_pair_offset, W_mlp1,
           b_mlp1, W_mlp2, b_mlp2, W_local, ln_local_scale, ln_local_offset):
    n = aa.shape[0]
    r_topk = min(512, n)
    r = min(512, n)
    k2 = 2 * NUM_NEIGH

    positions = all_atom_positions[:, 1]
    mask = all_atom_mask[:, 1] > 0
    posT = positions.T

    isaa_i32 = is_aa.astype(jnp.int32)
    chain_i32 = chain_index.astype(jnp.int32)
    resid_i32 = residue_index.astype(jnp.int32)
    mask_i32 = mask.astype(jnp.int32)
    aa_i32 = aa.astype(jnp.int32)

    cols_aux = jnp.stack([isaa_i32, mask_i32], axis=0)
    rows_aux = jnp.stack(
        [chain_i32, resid_i32, aa_i32, isaa_i32, mask_i32,
         jnp.zeros_like(aa_i32), jnp.zeros_like(aa_i32),
         jnp.zeros_like(aa_i32)], axis=1)

    nbr, wrap, dsel = _run_topk(positions, posT, cols_aux, rows_aux, n, r_topk)

    table = (resid_i32 << 4) | (chain_i32 << 1) | isaa_i32
    packed = _gather_packed(table, wrap.reshape(-1)).reshape(n, k2)

    pair_knm, local = _run_pair(
        dsel, packed, nbr, rows_aux, W_pair,
        ln_pair_scale.reshape(1, -1), ln_pair_offset.reshape(1, -1),
        W_mlp1, b_mlp1.reshape(1, -1), W_mlp2, b_mlp2.reshape(1, -1),
        W_local, ln_local_scale.reshape(1, -1), ln_local_offset.reshape(1, -1),
        n, r)

    return (local, jnp.transpose(pair_knm, (1, 0, 2)), nbr, mask)


# speedup vs baseline: 1.0130x; 1.0130x over previous
import functools

import jax
import jax.numpy as jnp
from jax import lax
from jax.experimental import pallas as pl
from jax.experimental.pallas import tpu as pltpu
from jax.experimental.pallas import tpu_sc as plsc

NUM_NEIGH = 16
RBF_BINS = 16
PAIR_SIZE = 64
LOCAL_SIZE = 128


def _topk_body(pos_ref, posT_ref, cols_ref, rows_ref, nbr_ref, wrap_ref,
               dsel_ref, *, n_total):
    xi = pos_ref[:, 0:1]
    yi = pos_ref[:, 1:2]
    zi = pos_ref[:, 2:3]
    xj = posT_ref[0:1, :]
    yj = posT_ref[1:2, :]
    zj = posT_ref[2:3, :]
    d2 = jnp.maximum((xi - xj) ** 2 + (yi - yj) ** 2 + (zi - zj) ** 2, 1e-12)

    isaa_j = cols_ref[0:1, :] > 0
    mask_j = cols_ref[1:2, :] > 0
    mask_i = rows_ref[:, 4:5] > 0

    inf = jnp.float32(jnp.inf)
    score_a = jnp.where(mask_i & mask_j & isaa_j, d2, inf)
    score_b = jnp.where(mask_i & mask_j & (~isaa_j), d2, inf)
    r = d2.shape[0]
    iota = lax.broadcasted_iota(jnp.int32, (r, n_total), 1)
    d2_last = d2[:, n_total - 1:n_total]

    for s, score in ((0, score_a), (1, score_b)):
        c0 = s * NUM_NEIGH
        for k in range(NUM_NEIGH):
            m = jnp.min(score, axis=1, keepdims=True)
            valid = m < inf
            idx = jnp.min(jnp.where(score == m, iota, n_total), axis=1,
                          keepdims=True)
            nbr_ref[:, c0 + k:c0 + k + 1] = jnp.where(valid, idx, -1)
            wrap_ref[:, c0 + k:c0 + k + 1] = jnp.where(valid, idx,
                                                       n_total - 1)
            dsel_ref[:, c0 + k:c0 + k + 1] = jnp.sqrt(
                jnp.where(valid, m, d2_last))
            score = jnp.where(iota == idx, inf, score)


def _run_topk(pos, posT, cols_aux, rows_aux, n, r):
    k2 = 2 * NUM_NEIGH
    grid = n // r
    return pl.pallas_call(
        functools.partial(_topk_body, n_total=n),
        grid=(grid,),
        in_specs=[
            pl.BlockSpec((r, 3), lambda i: (i, 0)),
            pl.BlockSpec((3, n), lambda i: (0, 0)),
            pl.BlockSpec((2, n), lambda i: (0, 0)),
            pl.BlockSpec((r, 8), lambda i: (i, 0)),
        ],
        out_specs=[
            pl.BlockSpec((r, k2), lambda i: (i, 0)),
            pl.BlockSpec((r, k2), lambda i: (i, 0)),
            pl.BlockSpec((r, k2), lambda i: (i, 0)),
        ],
        out_shape=[
            jax.ShapeDtypeStruct((n, k2), jnp.int32),
            jax.ShapeDtypeStruct((n, k2), jnp.int32),
            jax.ShapeDtypeStruct((n, k2), jnp.float32),
        ],
    )(pos, posT, cols_aux, rows_aux)


def _gather_packed(table, idx_flat):
    info = plsc.get_sparse_core_info()
    nw = info.num_cores * info.num_subcores
    b = idx_flat.shape[0]
    bw = b // nw
    n = table.shape[0]
    mesh = plsc.VectorSubcoreMesh(core_axis_name="c", subcore_axis_name="s")

    @functools.partial(
        pl.kernel,
        mesh=mesh,
        compiler_params=pltpu.CompilerParams(needs_layout_passes=False),
        out_type=jax.ShapeDtypeStruct((b,), jnp.int32),
        scratch_types=[
            pltpu.VMEM((n,), jnp.int32),
            pltpu.VMEM((bw,), jnp.int32),
            pltpu.VMEM((bw,), jnp.int32),
        ],
    )
    def gk(table_hbm, idx_hbm, out_hbm, table_v, idx_v, out_v):
        wid = lax.axis_index("s") * info.num_cores + lax.axis_index("c")
        base = wid * bw
        pltpu.sync_copy(table_hbm, table_v)
        pltpu.sync_copy(idx_hbm.at[pl.ds(base, bw)], idx_v)

        def body(i, carry):
            off = i * 16
            vec = idx_v[pl.ds(off, 16)]
            out_v[pl.ds(off, 16)] = plsc.load_gather(table_v, [vec])
            return carry

        lax.fori_loop(0, bw // 16, body, 0)
        pltpu.sync_copy(out_v, out_hbm.at[pl.ds(base, bw)])

    return gk(table, idx_flat)


def _pair_body(dsel_ref, packed_ref, nbr_ref, rows_ref, wp_ref, lps_ref,
               lpo_ref, w1_ref, b1_ref, w2_ref, b2_ref, wl_ref, lls_ref,
               llo_ref, pair_ref, local_ref, *, r):
    k2 = 2 * NUM_NEIGH

    chain_i = rows_ref[:, 0:1]
    resid_i = rows_ref[:, 1:2]
    aa_i = rows_ref[:, 2:3]
    isaa_i = rows_ref[:, 3:4]

    step = jnp.float32(20.0 / 15.0)
    centers = jnp.float32(2.0) + lax.broadcasted_iota(
        jnp.int32, (1, RBF_BINS), 1).astype(jnp.float32) * step
    sigma = jnp.float32(20.0 / 16.0)

    pj = packed_ref[...]
    isaa_j = (pj & 1).astype(jnp.float32)
    chain_j = (pj >> 1) & 7
    resid_j = pj >> 4
    same_chain = chain_i == chain_j
    other_chain = (~same_chain).astype(jnp.float32)
    same_res = (same_chain & (resid_i == resid_j)).astype(jnp.float32)
    nbm = (nbr_ref[...] != -1).astype(jnp.float32)

    d_rep = jnp.concatenate(
        [jnp.broadcast_to(dsel_ref[:, k:k + 1], (r, RBF_BINS))
         for k in range(k2)], axis=1)
    centers_rep = jnp.concatenate([centers] * k2, axis=1)
    rbf = jnp.exp(-((d_rep - centers_rep) ** 2) / (2.0 * sigma * sigma))

    rbf_stack = jnp.concatenate(
        [rbf[:, k * RBF_BINS:(k + 1) * RBF_BINS] for k in range(k2)], axis=0)
    col = lambda a: jnp.concatenate(
        [a[:, k:k + 1] for k in range(k2)], axis=0)
    isaa_s, same_s, other_s, nb = (col(isaa_j), col(same_res),
                                   col(other_chain), col(nbm))

    pair = (
        jnp.dot(rbf_stack, wp_ref[0:RBF_BINS, :],
                preferred_element_type=jnp.float32)
        + isaa_s * wp_ref[RBF_BINS:RBF_BINS + 1, :]
        + same_s * wp_ref[RBF_BINS + 1:RBF_BINS + 2, :]
        + other_s * wp_ref[RBF_BINS + 2:RBF_BINS + 3, :])
    mean = jnp.mean(pair, axis=1, keepdims=True)
    var = jnp.mean((pair - mean) ** 2, axis=1, keepdims=True)
    pair = (pair - mean) / jnp.sqrt(var + 1e-5) * lps_ref[...] + lpo_ref[...]
    pair_ref[...] = pair.reshape(k2, r, PAIR_SIZE)

    h = jnp.maximum(
        jnp.dot(pair, w1_ref[...], preferred_element_type=jnp.float32)
        + b1_ref[...], 0.0)
    h = jnp.dot(h, w2_ref[...], preferred_element_type=jnp.float32) + b2_ref[...]
    h = h * nb
    pw = jnp.sum(h.reshape(k2, r, LOCAL_SIZE), axis=0)

    onehot = (lax.broadcasted_iota(jnp.int32, (r, 21), 1) == aa_i).astype(
        jnp.float32)
    lfeat = jnp.concatenate([pw, isaa_i.astype(jnp.float32), onehot], axis=1)
    loc = jnp.dot(lfeat, wl_ref[...], preferred_element_type=jnp.float32)
    mean = jnp.mean(loc, axis=1, keepdims=True)
    var = jnp.mean((loc - mean) ** 2, axis=1, keepdims=True)
    loc = (loc - mean) / jnp.sqrt(var + 1e-5) * lls_ref[...] + llo_ref[...]
    local_ref[...] = loc


def _run_pair(dsel, packed, nbr, rows_aux, W_pair, lps, lpo, W1, b1, W2, b2,
              Wl, lls, llo, n, r):
    k2 = 2 * NUM_NEIGH
    grid = n // r
    full = lambda shape: pl.BlockSpec(shape, lambda i: tuple(0 for _ in shape))
    blk = lambda shape: pl.BlockSpec(shape, lambda i: (i, 0))
    return pl.pallas_call(
        functools.partial(_pair_body, r=r),
        grid=(grid,),
        in_specs=[
            blk((r, k2)), blk((r, k2)), blk((r, k2)), blk((r, 8)),
            full(W_pair.shape), full(lps.shape), full(lpo.shape),
            full(W1.shape), full(b1.shape), full(W2.shape), full(b2.shape),
            full(Wl.shape), full(lls.shape), full(llo.shape),
        ],
        out_specs=[
            pl.BlockSpec((k2, r, PAIR_SIZE), lambda i: (0, i, 0)),
            pl.BlockSpec((r, PAIR_SIZE), lambda i: (i, 0)),
        ],
        out_shape=[
            jax.ShapeDtypeStruct((k2, n, PAIR_SIZE), jnp.float32),
            jax.ShapeDtypeStruct((n, PAIR_SIZE), jnp.float32),
        ],
    )(dsel, packed, nbr, rows_aux, W_pair, lps, lpo, W1, b1, W2, b2,
      Wl, lls, llo)


def kernel(aa, chain_index, residue_index, is_aa, all_atom_positions,
           all_atom_mask, W_pair, ln_pair_scale, ln_pair_offset, W_mlp1,
           b_mlp1, W_mlp2, b_mlp2, W_local, ln_local_scale, ln_local_offset):
    n = aa.shape[0]
    r_topk = min(512, n)
    r = min(256, n)
    k2 = 2 * NUM_NEIGH

    positions = all_atom_positions[:, 1]
    mask = all_atom_mask[:, 1] > 0
    posT = positions.T

    isaa_i32 = is_aa.astype(jnp.int32)
    chain_i32 = chain_index.astype(jnp.int32)
    resid_i32 = residue_index.astype(jnp.int32)
    mask_i32 = mask.astype(jnp.int32)
    aa_i32 = aa.astype(jnp.int32)

    cols_aux = jnp.stack([isaa_i32, mask_i32], axis=0)
    rows_aux = jnp.stack(
        [chain_i32, resid_i32, aa_i32, isaa_i32, mask_i32,
         jnp.zeros_like(aa_i32), jnp.zeros_like(aa_i32),
         jnp.zeros_like(aa_i32)], axis=1)

    nbr, wrap, dsel = _run_topk(positions, posT, cols_aux, rows_aux, n, r_topk)

    table = (resid_i32 << 4) | (chain_i32 << 1) | isaa_i32
    packed = _gather_packed(table, wrap.reshape(-1)).reshape(n, k2)

    pair_knm, local = _run_pair(
        dsel, packed, nbr, rows_aux, W_pair,
        ln_pair_scale.reshape(1, -1), ln_pair_offset.reshape(1, -1),
        W_mlp1, b_mlp1.reshape(1, -1), W_mlp2, b_mlp2.reshape(1, -1),
        W_local, ln_local_scale.reshape(1, -1), ln_local_offset.reshape(1, -1),
        n, r)

    return (local, jnp.transpose(pair_knm, (1, 0, 2)), nbr, mask)


# speedup vs baseline: 1.0132x; 1.0001x over previous
import functools

import jax
import jax.numpy as jnp
from jax import lax
from jax.experimental import pallas as pl
from jax.experimental.pallas import tpu as pltpu
from jax.experimental.pallas import tpu_sc as plsc

NUM_NEIGH = 16
RBF_BINS = 16
PAIR_SIZE = 64
LOCAL_SIZE = 128


def _topk_body(pos_ref, posT_ref, cols_ref, rows_ref, nbr_ref, wrap_ref,
               dsel_ref, *, n_total):
    xi = pos_ref[:, 0:1]
    yi = pos_ref[:, 1:2]
    zi = pos_ref[:, 2:3]
    xj = posT_ref[0:1, :]
    yj = posT_ref[1:2, :]
    zj = posT_ref[2:3, :]
    d2 = jnp.maximum((xi - xj) ** 2 + (yi - yj) ** 2 + (zi - zj) ** 2, 1e-12)

    isaa_j = cols_ref[0:1, :] > 0
    mask_j = cols_ref[1:2, :] > 0
    mask_i = rows_ref[:, 4:5] > 0

    inf = jnp.float32(jnp.inf)
    score_a = jnp.where(mask_i & mask_j & isaa_j, d2, inf)
    score_b = jnp.where(mask_i & mask_j & (~isaa_j), d2, inf)
    r = d2.shape[0]
    iota = lax.broadcasted_iota(jnp.int32, (r, n_total), 1)
    d2_last = d2[:, n_total - 1:n_total]

    for s, score in ((0, score_a), (1, score_b)):
        c0 = s * NUM_NEIGH
        for k in range(NUM_NEIGH):
            m = jnp.min(score, axis=1, keepdims=True)
            valid = m < inf
            idx = jnp.min(jnp.where(score == m, iota, n_total), axis=1,
                          keepdims=True)
            nbr_ref[:, c0 + k:c0 + k + 1] = jnp.where(valid, idx, -1)
            wrap_ref[:, c0 + k:c0 + k + 1] = jnp.where(valid, idx,
                                                       n_total - 1)
            dsel_ref[:, c0 + k:c0 + k + 1] = jnp.sqrt(
                jnp.where(valid, m, d2_last))
            score = jnp.where(iota == idx, inf, score)


def _run_topk(pos, posT, cols_aux, rows_aux, n, r):
    k2 = 2 * NUM_NEIGH
    grid = n // r
    return pl.pallas_call(
        functools.partial(_topk_body, n_total=n),
        grid=(grid,),
        compiler_params=pltpu.CompilerParams(
            dimension_semantics=("parallel",)),
        in_specs=[
            pl.BlockSpec((r, 3), lambda i: (i, 0)),
            pl.BlockSpec((3, n), lambda i: (0, 0)),
            pl.BlockSpec((2, n), lambda i: (0, 0)),
            pl.BlockSpec((r, 8), lambda i: (i, 0)),
        ],
        out_specs=[
            pl.BlockSpec((r, k2), lambda i: (i, 0)),
            pl.BlockSpec((r, k2), lambda i: (i, 0)),
            pl.BlockSpec((r, k2), lambda i: (i, 0)),
        ],
        out_shape=[
            jax.ShapeDtypeStruct((n, k2), jnp.int32),
            jax.ShapeDtypeStruct((n, k2), jnp.int32),
            jax.ShapeDtypeStruct((n, k2), jnp.float32),
        ],
    )(pos, posT, cols_aux, rows_aux)


def _gather_packed(table, idx_flat):
    info = plsc.get_sparse_core_info()
    nw = info.num_cores * info.num_subcores
    b = idx_flat.shape[0]
    bw = b // nw
    n = table.shape[0]
    mesh = plsc.VectorSubcoreMesh(core_axis_name="c", subcore_axis_name="s")

    @functools.partial(
        pl.kernel,
        mesh=mesh,
        compiler_params=pltpu.CompilerParams(needs_layout_passes=False),
        out_type=jax.ShapeDtypeStruct((b,), jnp.int32),
        scratch_types=[
            pltpu.VMEM((n,), jnp.int32),
            pltpu.VMEM((bw,), jnp.int32),
            pltpu.VMEM((bw,), jnp.int32),
        ],
    )
    def gk(table_hbm, idx_hbm, out_hbm, table_v, idx_v, out_v):
        wid = lax.axis_index("s") * info.num_cores + lax.axis_index("c")
        base = wid * bw
        pltpu.sync_copy(table_hbm, table_v)
        pltpu.sync_copy(idx_hbm.at[pl.ds(base, bw)], idx_v)

        def body(i, carry):
            off = i * 16
            vec = idx_v[pl.ds(off, 16)]
            out_v[pl.ds(off, 16)] = plsc.load_gather(table_v, [vec])
            return carry

        lax.fori_loop(0, bw // 16, body, 0)
        pltpu.sync_copy(out_v, out_hbm.at[pl.ds(base, bw)])

    return gk(table, idx_flat)


def _pair_body(dsel_ref, packed_ref, nbr_ref, rows_ref, wp_ref, lps_ref,
               lpo_ref, w1_ref, b1_ref, w2_ref, b2_ref, wl_ref, lls_ref,
               llo_ref, pair_ref, local_ref, *, r):
    k2 = 2 * NUM_NEIGH

    chain_i = rows_ref[:, 0:1]
    resid_i = rows_ref[:, 1:2]
    aa_i = rows_ref[:, 2:3]
    isaa_i = rows_ref[:, 3:4]

    step = jnp.float32(20.0 / 15.0)
    centers = jnp.float32(2.0) + lax.broadcasted_iota(
        jnp.int32, (1, RBF_BINS), 1).astype(jnp.float32) * step
    sigma = jnp.float32(20.0 / 16.0)

    pj = packed_ref[...]
    isaa_j = (pj & 1).astype(jnp.float32)
    chain_j = (pj >> 1) & 7
    resid_j = pj >> 4
    same_chain = chain_i == chain_j
    other_chain = (~same_chain).astype(jnp.float32)
    same_res = (same_chain & (resid_i == resid_j)).astype(jnp.float32)
    nbm = (nbr_ref[...] != -1).astype(jnp.float32)

    d_rep = jnp.concatenate(
        [jnp.broadcast_to(dsel_ref[:, k:k + 1], (r, RBF_BINS))
         for k in range(k2)], axis=1)
    centers_rep = jnp.concatenate([centers] * k2, axis=1)
    rbf = jnp.exp(-((d_rep - centers_rep) ** 2) / (2.0 * sigma * sigma))

    rbf_stack = jnp.concatenate(
        [rbf[:, k * RBF_BINS:(k + 1) * RBF_BINS] for k in range(k2)], axis=0)
    col = lambda a: jnp.concatenate(
        [a[:, k:k + 1] for k in range(k2)], axis=0)
    isaa_s, same_s, other_s, nb = (col(isaa_j), col(same_res),
                                   col(other_chain), col(nbm))

    pair = (
        jnp.dot(rbf_stack, wp_ref[0:RBF_BINS, :],
                preferred_element_type=jnp.float32)
        + isaa_s * wp_ref[RBF_BINS:RBF_BINS + 1, :]
        + same_s * wp_ref[RBF_BINS + 1:RBF_BINS + 2, :]
        + other_s * wp_ref[RBF_BINS + 2:RBF_BINS + 3, :])
    mean = jnp.mean(pair, axis=1, keepdims=True)
    var = jnp.mean((pair - mean) ** 2, axis=1, keepdims=True)
    pair = (pair - mean) / jnp.sqrt(var + 1e-5) * lps_ref[...] + lpo_ref[...]
    pair_ref[...] = pair.reshape(k2, r, PAIR_SIZE)

    h = jnp.maximum(
        jnp.dot(pair, w1_ref[...], preferred_element_type=jnp.float32)
        + b1_ref[...], 0.0)
    h = jnp.dot(h, w2_ref[...], preferred_element_type=jnp.float32) + b2_ref[...]
    h = h * nb
    pw = jnp.sum(h.reshape(k2, r, LOCAL_SIZE), axis=0)

    onehot = (lax.broadcasted_iota(jnp.int32, (r, 21), 1) == aa_i).astype(
        jnp.float32)
    lfeat = jnp.concatenate([pw, isaa_i.astype(jnp.float32), onehot], axis=1)
    loc = jnp.dot(lfeat, wl_ref[...], preferred_element_type=jnp.float32)
    mean = jnp.mean(loc, axis=1, keepdims=True)
    var = jnp.mean((loc - mean) ** 2, axis=1, keepdims=True)
    loc = (loc - mean) / jnp.sqrt(var + 1e-5) * lls_ref[...] + llo_ref[...]
    local_ref[...] = loc


def _run_pair(dsel, packed, nbr, rows_aux, W_pair, lps, lpo, W1, b1, W2, b2,
              Wl, lls, llo, n, r):
    k2 = 2 * NUM_NEIGH
    grid = n // r
    full = lambda shape: pl.BlockSpec(shape, lambda i: tuple(0 for _ in shape))
    blk = lambda shape: pl.BlockSpec(shape, lambda i: (i, 0))
    return pl.pallas_call(
        functools.partial(_pair_body, r=r),
        grid=(grid,),
        compiler_params=pltpu.CompilerParams(
            dimension_semantics=("parallel",)),
        in_specs=[
            blk((r, k2)), blk((r, k2)), blk((r, k2)), blk((r, 8)),
            full(W_pair.shape), full(lps.shape), full(lpo.shape),
            full(W1.shape), full(b1.shape), full(W2.shape), full(b2.shape),
            full(Wl.shape), full(lls.shape), full(llo.shape),
        ],
        out_specs=[
            pl.BlockSpec((k2, r, PAIR_SIZE), lambda i: (0, i, 0)),
            pl.BlockSpec((r, PAIR_SIZE), lambda i: (i, 0)),
        ],
        out_shape=[
            jax.ShapeDtypeStruct((k2, n, PAIR_SIZE), jnp.float32),
            jax.ShapeDtypeStruct((n, PAIR_SIZE), jnp.float32),
        ],
    )(dsel, packed, nbr, rows_aux, W_pair, lps, lpo, W1, b1, W2, b2,
      Wl, lls, llo)


def kernel(aa, chain_index, residue_index, is_aa, all_atom_positions,
           all_atom_mask, W_pair, ln_pair_scale, ln_pair_offset, W_mlp1,
           b_mlp1, W_mlp2, b_mlp2, W_local, ln_local_scale, ln_local_offset):
    n = aa.shape[0]
    r_topk = min(512, n)
    r = min(256, n)
    k2 = 2 * NUM_NEIGH

    positions = all_atom_positions[:, 1]
    mask = all_atom_mask[:, 1] > 0
    posT = positions.T

    isaa_i32 = is_aa.astype(jnp.int32)
    chain_i32 = chain_index.astype(jnp.int32)
    resid_i32 = residue_index.astype(jnp.int32)
    mask_i32 = mask.astype(jnp.int32)
    aa_i32 = aa.astype(jnp.int32)

    cols_aux = jnp.stack([isaa_i32, mask_i32], axis=0)
    rows_aux = jnp.stack(
        [chain_i32, resid_i32, aa_i32, isaa_i32, mask_i32,
         jnp.zeros_like(aa_i32), jnp.zeros_like(aa_i32),
         jnp.zeros_like(aa_i32)], axis=1)

    nbr, wrap, dsel = _run_topk(positions, posT, cols_aux, rows_aux, n, r_topk)

    table = (resid_i32 << 4) | (chain_i32 << 1) | isaa_i32
    packed = _gather_packed(table, wrap.reshape(-1)).reshape(n, k2)

    pair_knm, local = _run_pair(
        dsel, packed, nbr, rows_aux, W_pair,
        ln_pair_scale.reshape(1, -1), ln_pair_offset.reshape(1, -1),
        W_mlp1, b_mlp1.reshape(1, -1), W_mlp2, b_mlp2.reshape(1, -1),
        W_local, ln_local_scale.reshape(1, -1), ln_local_offset.reshape(1, -1),
        n, r)

    return (local, jnp.transpose(pair_knm, (1, 0, 2)), nbr, mask)
